# baseline (speedup 1.0000x reference)
import functools

import jax
import jax.numpy as jnp
from jax import lax
from jax.experimental import pallas as pl
from jax.experimental.pallas import tpu as pltpu

_NIN = 3
_NOUT = 3


def _manual_body(x_hbm, w1_hbm, w2_hbm, b2_hbm, o_hbm,
                 xbuf, obuf, w1s, w2s, b2s, hs,
                 load_sem, store_sem, w_sem, *, tm, tiles_per_core):
    core = pl.program_id(0)
    base = core * tiles_per_core
    rank_p = w1s.shape[1]

    w1c = pltpu.make_async_copy(w1_hbm, w1s, w_sem)
    w2c = pltpu.make_async_copy(w2_hbm, w2s.at[pl.ds(0, rank_p)], w_sem)
    b2c = pltpu.make_async_copy(b2_hbm, b2s, w_sem)
    w1c.start()
    w2c.start()
    b2c.start()

    def start_load(t, slot):
        pltpu.make_async_copy(
            x_hbm.at[pl.ds((base + t) * tm, tm), :],
            xbuf.at[slot],
            load_sem.at[slot],
        ).start()

    for s in range(_NIN):
        start_load(s, s)

    ones_col = (lax.broadcasted_iota(jnp.int32, (tm, 2 * rank_p), 1)
                == rank_p)
    hs[...] = ones_col.astype(hs.dtype)
    w1c.wait()
    w2c.wait()
    b2c.wait()
    w2s[pl.ds(rank_p, rank_p), :] = jnp.zeros_like(w2s[pl.ds(rank_p, rank_p), :])
    w2s[pl.ds(rank_p, 1), :] = b2s[...].astype(w2s.dtype)

    def step(t, carry):
        slot = lax.rem(t, _NIN)
        oslot = lax.rem(t, _NOUT)
        pltpu.make_async_copy(
            x_hbm.at[pl.ds(0, tm), :], xbuf.at[slot], load_sem.at[slot]
        ).wait()

        @pl.when(t >= _NOUT)
        def _():
            pltpu.make_async_copy(
                obuf.at[oslot], o_hbm.at[pl.ds(0, tm), :], store_sem.at[oslot]
            ).wait()

        h = jnp.dot(xbuf[slot], w1s[...], preferred_element_type=jnp.float32)
        hs[:, pl.ds(0, rank_p)] = h.astype(hs.dtype)
        obuf[oslot] = jnp.dot(hs[...], w2s[...],
                              preferred_element_type=jnp.float32
                              ).astype(obuf.dtype)

        pltpu.make_async_copy(
            obuf.at[oslot],
            o_hbm.at[pl.ds((base + t) * tm, tm), :],
            store_sem.at[oslot],
        ).start()

        @pl.when(t + _NIN < tiles_per_core)
        def _():
            start_load(t + _NIN, slot)

        return carry

    lax.fori_loop(0, tiles_per_core, step, 0)

    def drain(t, carry):
        oslot = lax.rem(t, _NOUT)
        pltpu.make_async_copy(
            obuf.at[oslot], o_hbm.at[pl.ds(0, tm), :], store_sem.at[oslot]
        ).wait()
        return carry

    lax.fori_loop(max(tiles_per_core - _NOUT, 0), tiles_per_core, drain, 0)


@functools.partial(jax.jit, static_argnames=("tm", "interpret"))
def _manual_call(x, w1t, w2t, b2p, tm, interpret=False):
    B, d_in = x.shape
    rank_p = w1t.shape[1]
    d_out_p = w2t.shape[1]
    n_cores = 2
    tiles_per_core = B // (n_cores * tm)
    body = functools.partial(_manual_body, tm=tm,
                             tiles_per_core=tiles_per_core)
    return pl.pallas_call(
        body,
        out_shape=jax.ShapeDtypeStruct((B, d_out_p), jnp.bfloat16),
        grid=(n_cores,),
        in_specs=[
            pl.BlockSpec(memory_space=pl.ANY),
            pl.BlockSpec(memory_space=pl.ANY),
            pl.BlockSpec(memory_space=pl.ANY),
            pl.BlockSpec(memory_space=pl.ANY),
        ],
        out_specs=pl.BlockSpec(memory_space=pl.ANY),
        scratch_shapes=[
            pltpu.VMEM((_NIN, tm, d_in), jnp.bfloat16),
            pltpu.VMEM((_NOUT, tm, d_out_p), jnp.bfloat16),
            pltpu.VMEM((d_in, rank_p), jnp.bfloat16),
            pltpu.VMEM((2 * rank_p, d_out_p), jnp.bfloat16),
            pltpu.VMEM((1, d_out_p), jnp.float32),
            pltpu.VMEM((tm, 2 * rank_p), jnp.bfloat16),
            pltpu.SemaphoreType.DMA((_NIN,)),
            pltpu.SemaphoreType.DMA((_NOUT,)),
            pltpu.SemaphoreType.DMA,
        ],
        compiler_params=pltpu.CompilerParams(
            dimension_semantics=("parallel",),
            vmem_limit_bytes=100 * 1024 * 1024,
        ),
        interpret=interpret,
    )(x, w1t, w2t, b2p)


def _emitter_body(xa_ref, xb_ref, w1a_ref, w1b_ref, w2t_ref, b2_ref, o_ref):
    h = jnp.dot(xa_ref[...], w1a_ref[...], preferred_element_type=jnp.float32)
    h = h + jnp.dot(xb_ref[...], w1b_ref[...],
                    preferred_element_type=jnp.float32)
    y = jnp.dot(h.astype(w2t_ref.dtype), w2t_ref[...],
                preferred_element_type=jnp.float32)
    o_ref[...] = (y + b2_ref[...]).astype(o_ref.dtype)


@functools.partial(jax.jit, static_argnames=("tm",))
def _emitter_call(x, w1t, w2t, b2p, tm):
    B, d_in = x.shape
    rank_p = w1t.shape[1]
    d_out_p = w2t.shape[1]
    d2 = d_in // 2
    return pl.pallas_call(
        _emitter_body,
        out_shape=jax.ShapeDtypeStruct((B, d_out_p), jnp.bfloat16),
        grid=(pl.cdiv(B, tm),),
        in_specs=[
            pl.BlockSpec((tm, d2), lambda i: (i, 0)),
            pl.BlockSpec((tm, d2), lambda i: (i, 1)),
            pl.BlockSpec((d2, rank_p), lambda i: (0, 0)),
            pl.BlockSpec((d2, rank_p), lambda i: (1, 0)),
            pl.BlockSpec((rank_p, d_out_p), lambda i: (0, 0)),
            pl.BlockSpec((1, d_out_p), lambda i: (0, 0)),
        ],
        out_specs=pl.BlockSpec((tm, d_out_p), lambda i: (i, 0)),
        compiler_params=pltpu.CompilerParams(
            dimension_semantics=("parallel",),
            vmem_limit_bytes=100 * 1024 * 1024,
        ),
    )(x, x, w1t, w1t, w2t, b2p)


def kernel(x, w1t, w2t, b2p):
    B = x.shape[0]
    tm = 256
    x = x if x.dtype == w1t.dtype else x.astype(w1t.dtype)
    if B % (2 * tm) == 0 and B // (2 * tm) >= max(_NIN, _NOUT):
        return _manual_call(x, w1t, w2t, b2p, tm)
    while tm > 8 and B % tm:
        tm //= 2
    return _emitter_call(x, w1t, w2t, b2p, max(tm, 8))


# speedup vs baseline: 1.0147x; 1.0147x over previous
import functools

import jax
import jax.numpy as jnp
from jax import lax
from jax.experimental import pallas as pl
from jax.experimental.pallas import tpu as pltpu

_NIN = 3
_NOUT = 3


def _manual_body(x_hbm, w1_hbm, w2_hbm, b2_hbm, o_hbm,
                 xbuf, obuf, w1s, w2s, b2s,
                 load_sem, store_sem, w_sem, *, tm, tiles_per_core):
    core = pl.program_id(0)
    base = core * tiles_per_core
    rank_p = w1s.shape[1]

    w1c = pltpu.make_async_copy(w1_hbm, w1s, w_sem)
    w2c = pltpu.make_async_copy(w2_hbm, w2s, w_sem)
    b2c = pltpu.make_async_copy(b2_hbm, b2s, w_sem)
    w1c.start()
    w2c.start()
    b2c.start()

    def start_load(t, slot):
        pltpu.make_async_copy(
            x_hbm.at[pl.ds((base + t) * tm, tm), :],
            xbuf.at[slot],
            load_sem.at[slot],
        ).start()

    for s in range(_NIN):
        start_load(s, s)

    w1c.wait()
    w2c.wait()
    b2c.wait()

    def step(t, carry):
        slot = lax.rem(t, _NIN)
        oslot = lax.rem(t, _NOUT)
        pltpu.make_async_copy(
            x_hbm.at[pl.ds(0, tm), :], xbuf.at[slot], load_sem.at[slot]
        ).wait()

        @pl.when(t >= _NOUT)
        def _():
            pltpu.make_async_copy(
                obuf.at[oslot], o_hbm.at[pl.ds(0, tm), :], store_sem.at[oslot]
            ).wait()

        h = jnp.dot(xbuf[slot], w1s[...], preferred_element_type=jnp.float32)
        y = jnp.dot(h.astype(w2s.dtype), w2s[...],
                    preferred_element_type=jnp.float32)
        obuf[oslot] = (y + b2s[...]).astype(obuf.dtype)

        pltpu.make_async_copy(
            obuf.at[oslot],
            o_hbm.at[pl.ds((base + t) * tm, tm), :],
            store_sem.at[oslot],
        ).start()

        @pl.when(t + _NIN < tiles_per_core)
        def _():
            start_load(t + _NIN, slot)

        return carry

    lax.fori_loop(0, tiles_per_core, step, 0, unroll=2)

    def drain(t, carry):
        oslot = lax.rem(t, _NOUT)
        pltpu.make_async_copy(
            obuf.at[oslot], o_hbm.at[pl.ds(0, tm), :], store_sem.at[oslot]
        ).wait()
        return carry

    lax.fori_loop(max(tiles_per_core - _NOUT, 0), tiles_per_core, drain, 0)


@functools.partial(jax.jit, static_argnames=("tm", "interpret"))
def _manual_call(x, w1t, w2t, b2p, tm, interpret=False):
    B, d_in = x.shape
    rank_p = w1t.shape[1]
    d_out_p = w2t.shape[1]
    n_cores = 2
    tiles_per_core = B // (n_cores * tm)
    body = functools.partial(_manual_body, tm=tm,
                             tiles_per_core=tiles_per_core)
    return pl.pallas_call(
        body,
        out_shape=jax.ShapeDtypeStruct((B, d_out_p), jnp.bfloat16),
        grid=(n_cores,),
        in_specs=[
            pl.BlockSpec(memory_space=pl.ANY),
            pl.BlockSpec(memory_space=pl.ANY),
            pl.BlockSpec(memory_space=pl.ANY),
            pl.BlockSpec(memory_space=pl.ANY),
        ],
        out_specs=pl.BlockSpec(memory_space=pl.ANY),
        scratch_shapes=[
            pltpu.VMEM((_NIN, tm, d_in), jnp.bfloat16),
            pltpu.VMEM((_NOUT, tm, d_out_p), jnp.bfloat16),
            pltpu.VMEM((d_in, rank_p), jnp.bfloat16),
            pltpu.VMEM((rank_p, d_out_p), jnp.bfloat16),
            pltpu.VMEM((1, d_out_p), jnp.float32),
            pltpu.SemaphoreType.DMA((_NIN,)),
            pltpu.SemaphoreType.DMA((_NOUT,)),
            pltpu.SemaphoreType.DMA,
        ],
        compiler_params=pltpu.CompilerParams(
            dimension_semantics=("parallel",),
            vmem_limit_bytes=100 * 1024 * 1024,
        ),
        interpret=interpret,
    )(x, w1t, w2t, b2p)


def _emitter_body(xa_ref, xb_ref, w1a_ref, w1b_ref, w2t_ref, b2_ref, o_ref):
    h = jnp.dot(xa_ref[...], w1a_ref[...], preferred_element_type=jnp.float32)
    h = h + jnp.dot(xb_ref[...], w1b_ref[...],
                    preferred_element_type=jnp.float32)
    y = jnp.dot(h.astype(w2t_ref.dtype), w2t_ref[...],
                preferred_element_type=jnp.float32)
    o_ref[...] = (y + b2_ref[...]).astype(o_ref.dtype)


@functools.partial(jax.jit, static_argnames=("tm",))
def _emitter_call(x, w1t, w2t, b2p, tm):
    B, d_in = x.shape
    rank_p = w1t.shape[1]
    d_out_p = w2t.shape[1]
    d2 = d_in // 2
    return pl.pallas_call(
        _emitter_body,
        out_shape=jax.ShapeDtypeStruct((B, d_out_p), jnp.bfloat16),
        grid=(pl.cdiv(B, tm),),
        in_specs=[
            pl.BlockSpec((tm, d2), lambda i: (i, 0)),
            pl.BlockSpec((tm, d2), lambda i: (i, 1)),
            pl.BlockSpec((d2, rank_p), lambda i: (0, 0)),
            pl.BlockSpec((d2, rank_p), lambda i: (1, 0)),
            pl.BlockSpec((rank_p, d_out_p), lambda i: (0, 0)),
            pl.BlockSpec((1, d_out_p), lambda i: (0, 0)),
        ],
        out_specs=pl.BlockSpec((tm, d_out_p), lambda i: (i, 0)),
        compiler_params=pltpu.CompilerParams(
            dimension_semantics=("parallel",),
            vmem_limit_bytes=100 * 1024 * 1024,
        ),
    )(x, x, w1t, w1t, w2t, b2p)


def kernel(x, w1t, w2t, b2p):
    B = x.shape[0]
    tm = 256
    x = x if x.dtype == w1t.dtype else x.astype(w1t.dtype)
    if B % (2 * tm) == 0 and B // (2 * tm) >= max(_NIN, _NOUT):
        return _manual_call(x, w1t, w2t, b2p, tm)
    while tm > 8 and B % tm:
        tm //= 2
    return _emitter_call(x, w1t, w2t, b2p, max(tm, 8))
